# baseline (speedup 1.0000x reference)
import functools

import jax
import jax.numpy as jnp
from jax import lax
from jax.experimental import pallas as pl
from jax.experimental.pallas import tpu as pltpu
from jax.experimental.pallas import tpu_sc as plsc

N_NODES = 10000
N_EDGES = 320000
D_FEAT = 128

_NC, _NS = 2, 16
_NW = _NC * _NS
_E_PER = N_EDGES // _NW
_CHUNK = 16
_NPT = 320

_mesh = plsc.VectorSubcoreMesh(core_axis_name="c", subcore_axis_name="s")


@functools.partial(
    pl.kernel,
    mesh=_mesh,
    out_type=[
        jax.ShapeDtypeStruct((N_NODES,), jnp.float32),
        jax.ShapeDtypeStruct((N_NODES,), jnp.float32),
    ],
    scratch_types=[
        pltpu.VMEM((_NPT, D_FEAT), jnp.float32),
        pltpu.VMEM((2 * D_FEAT,), jnp.float32),
        pltpu.VMEM((_CHUNK,), jnp.float32),
        pltpu.VMEM((_NPT,), jnp.float32),
        pltpu.VMEM((_NPT,), jnp.float32),
        pltpu.SemaphoreType.DMA,
    ],
    compiler_params=pltpu.CompilerParams(needs_layout_passes=False),
)
def _build_scores(
    x_hbm, w_hbm, b_hbm, s0_hbm, s1_hbm, x_v, w_v, b_v, s0_v, s1_v, sem
):
    wid = lax.axis_index("s") * _NC + lax.axis_index("c")
    base = jnp.minimum(wid * _NPT, N_NODES - _NPT)
    cp = pltpu.async_copy(x_hbm.at[pl.ds(base, _NPT)], x_v, sem)
    pltpu.sync_copy(w_hbm, w_v)
    pltpu.sync_copy(b_hbm, b_v)
    cp.wait()

    w0 = [w_v[pl.ds(c * _CHUNK, _CHUNK)] for c in range(D_FEAT // _CHUNK)]
    w1 = [
        w_v[pl.ds(D_FEAT + c * _CHUNK, _CHUNK)] for c in range(D_FEAT // _CHUNK)
    ]
    b_vec = b_v[...]
    lanes = lax.iota(jnp.int32, _CHUNK)

    @plsc.parallel_loop(0, _NPT // _CHUNK, 1, unroll=1)
    def _group(g):
        packed0 = b_vec
        packed1 = jnp.zeros((_CHUNK,), jnp.float32)
        for j in range(_CHUNK):
            n = g * _CHUNK + j
            acc0 = jnp.zeros((_CHUNK,), jnp.float32)
            acc1 = jnp.zeros((_CHUNK,), jnp.float32)
            for c in range(D_FEAT // _CHUNK):
                xa = x_v[n, pl.ds(c * _CHUNK, _CHUNK)]
                acc0 = acc0 + xa * w0[c]
                acc1 = acc1 + xa * w1[c]
            packed0 = jnp.where(lanes == j, packed0 + jnp.sum(acc0), packed0)
            packed1 = jnp.where(lanes == j, packed1 + jnp.sum(acc1), packed1)
        off = pl.multiple_of(g * _CHUNK, _CHUNK)
        s0_v[pl.ds(off, _CHUNK)] = packed0
        s1_v[pl.ds(off, _CHUNK)] = packed1

    pltpu.sync_copy(s0_v, s0_hbm.at[pl.ds(base, _NPT)])
    pltpu.sync_copy(s1_v, s1_hbm.at[pl.ds(base, _NPT)])


@functools.partial(
    pl.kernel,
    mesh=_mesh,
    out_type=jax.ShapeDtypeStruct((N_EDGES,), jnp.float32),
    scratch_types=[
        pltpu.VMEM((N_NODES,), jnp.float32),
        pltpu.VMEM((N_NODES,), jnp.float32),
        pltpu.VMEM((_E_PER,), jnp.int32),
        pltpu.VMEM((_E_PER,), jnp.int32),
        pltpu.VMEM((_E_PER,), jnp.float32),
        pltpu.SemaphoreType.DMA,
        pltpu.SemaphoreType.DMA,
        pltpu.SemaphoreType.DMA,
        pltpu.SemaphoreType.DMA,
    ],
    compiler_params=pltpu.CompilerParams(needs_layout_passes=False),
)
def _edge_gather(
    s0_hbm, s1_hbm, src_hbm, tgt_hbm, out_hbm,
    s0_v, s1_v, src_v, tgt_v, out_v, sem0, sem1, sem2, sem3,
):
    wid = lax.axis_index("s") * _NC + lax.axis_index("c")
    base = wid * _E_PER
    cp0 = pltpu.async_copy(s0_hbm, s0_v, sem0)
    cp1 = pltpu.async_copy(s1_hbm, s1_v, sem1)
    cp2 = pltpu.async_copy(src_hbm.at[pl.ds(base, _E_PER)], src_v, sem2)
    cp3 = pltpu.async_copy(tgt_hbm.at[pl.ds(base, _E_PER)], tgt_v, sem3)
    cp0.wait()
    cp1.wait()
    cp2.wait()
    cp3.wait()

    @plsc.parallel_loop(0, _E_PER // _CHUNK, 1, unroll=8)
    def _loop(i):
        off = pl.multiple_of(i * _CHUNK, _CHUNK)
        si = src_v[pl.ds(off, _CHUNK)]
        ti = tgt_v[pl.ds(off, _CHUNK)]
        vs = plsc.load_gather(s0_v, [si])
        vt = plsc.load_gather(s1_v, [ti])
        out_v[pl.ds(off, _CHUNK)] = vs + vt

    pltpu.sync_copy(out_v, out_hbm.at[pl.ds(base, _E_PER)])


def kernel(source_nodes, target_nodes, node_features, W, b):
    src = source_nodes.astype(jnp.int32)
    tgt = target_nodes.astype(jnp.int32)
    w_flat = W.reshape(-1)
    b16 = jnp.broadcast_to(b, (_CHUNK,))
    s0, s1 = _build_scores(node_features, w_flat, b16)
    return _edge_gather(s0, s1, src, tgt)
